# baseline (speedup 1.0000x reference)
import functools

import jax
import jax.numpy as jnp
from jax import lax
from jax.experimental import pallas as pl
from jax.experimental.pallas import tpu as pltpu
from jax.experimental.pallas import tpu_sc as plsc

_VOCAB = 100000
_DIM = 64
_B = 4096
_L = 200
_TAGS = 59
_TDIM = 4
_EDIM = _TDIM + 2
_EPAD = 8
_ODIM = _DIM + _EDIM
_EXT = _TAGS * 4

_N = _B * _L
_LANES = 256
_ROWS = _N // _LANES
_NC = 2
_NS = 16
_NW = _NC * _NS
_ROWS_PER_W = _ROWS // _NW
_CHUNK_ROWS = 2
_CHUNK = _CHUNK_ROWS * _LANES
_STEPS = _ROWS_PER_W // _CHUNK_ROWS
_CPL = _B // _CHUNK


def _sc_gather(word_table, ext_table, idx_all):
    mesh = plsc.VectorSubcoreMesh(core_axis_name="c", subcore_axis_name="s")

    @functools.partial(
        pl.kernel,
        mesh=mesh,
        compiler_params=pltpu.CompilerParams(
            use_tc_tiling_on_sc=False, needs_layout_passes=False),
        out_type=jax.ShapeDtypeStruct((_ODIM, _L, _B), jnp.float32),
        scratch_types=[
            pltpu.VMEM((_CHUNK_ROWS, 2, _LANES), jnp.int32),
            pltpu.VMEM((_CHUNK_ROWS, 2, _LANES), jnp.int32),
            pltpu.VMEM((_CHUNK, _DIM), jnp.float32),
            pltpu.VMEM((_CHUNK, _DIM), jnp.float32),
            pltpu.VMEM((_ODIM, 1, _CHUNK), jnp.float32),
            pltpu.VMEM((_EXT * _EPAD,), jnp.float32),
            pltpu.SemaphoreType.DMA,
            pltpu.SemaphoreType.DMA,
            pltpu.SemaphoreType.DMA,
        ],
    )
    def k(word_hbm, ext_hbm, idx_hbm, out_hbm,
          ibuf0, ibuf1, wbuf0, wbuf1, tbuf, extv, gsem0, gsem1, ssem):
        wid = lax.axis_index("s") * _NC + lax.axis_index("c")
        row0 = wid * _ROWS_PER_W
        cg0 = wid * _STEPS
        ibuf = (ibuf0, ibuf1)
        wbuf = (wbuf0, wbuf1)
        gsem = (gsem0, gsem1)
        lane = lax.iota(jnp.int32, 16)
        lane64 = lane * _DIM

        def gather_copies(p):
            for b in range(_CHUNK_ROWS):
                yield pltpu.make_async_copy(
                    word_hbm.at[ibuf[p].at[b, 0]],
                    wbuf[p].at[pl.ds(b * _LANES, _LANES)],
                    gsem[p])

        def store_copy(cg):
            l = cg // _CPL
            b0 = (cg % _CPL) * _CHUNK
            return pltpu.make_async_copy(
                tbuf,
                out_hbm.at[:, pl.ds(l, 1), pl.ds(b0, _CHUNK)],
                ssem)

        def transpose_ext(p):
            def plane(c, carry):
                col = jnp.full((16,), 0, jnp.int32) + c
                for g in range(_CHUNK // 16):
                    vals = plsc.load_gather(wbuf[p], [lane + g * 16, col])
                    tbuf[c, 0, pl.ds(g * 16, 16)] = vals
                return carry

            lax.fori_loop(0, _DIM, plane, 0)
            for b in range(_CHUNK_ROWS):
                for g in range(_LANES // 16):
                    i0 = b * _LANES + g * 16
                    e8 = ibuf[p][b, 1, pl.ds(g * 16, 16)] * _EPAD
                    for c in range(_EDIM):
                        vals = plsc.load_gather(extv, [e8 + c])
                        tbuf[_DIM + c, 0, pl.ds(i0, 16)] = vals

        pltpu.sync_copy(ext_hbm, extv)
        pltpu.sync_copy(idx_hbm.at[pl.ds(row0, _CHUNK_ROWS)], ibuf[0])
        for c in gather_copies(0):
            c.start()
        pltpu.sync_copy(
            idx_hbm.at[pl.ds(row0 + _CHUNK_ROWS, _CHUNK_ROWS)], ibuf[1])

        def step(j, p):
            cg = cg0 + j

            @pl.when(j >= 1)
            def _():
                store_copy(cg).wait()

            for c in gather_copies(p):
                c.wait()

            @pl.when(j + 1 < _STEPS)
            def _():
                for c in gather_copies(1 - p):
                    c.start()

            transpose_ext(p)
            store_copy(cg).start()

            @pl.when(j + 2 < _STEPS)
            def _():
                pltpu.sync_copy(
                    idx_hbm.at[pl.ds(row0 + (j + 2) * _CHUNK_ROWS,
                                     _CHUNK_ROWS)],
                    ibuf[p])

        def body(i, carry):
            step(2 * i, 0)
            step(2 * i + 1, 1)
            return carry

        lax.fori_loop(0, _STEPS // 2, body, 0)
        store_copy(cg0 + _STEPS - 1).wait()

    return k(word_table, ext_table, idx_all)


def _tc_prep(data_0, data_1, data_2, data_3):
    bb = _LANES

    def body(d0, d1, d2, d3, o):
        e = d1[...] * 4 + d2[...] * 2 + d3[...]
        o[:, 0, 0, :] = jnp.transpose(d0[...], (1, 0))
        o[:, 0, 1, :] = jnp.transpose(e, (1, 0))

    out = pl.pallas_call(
        body,
        grid=(_B // bb,),
        in_specs=[pl.BlockSpec((bb, _L), lambda j: (j, 0))] * 4,
        out_specs=pl.BlockSpec((_L, 1, 2, _LANES), lambda j: (0, j, 0, 0)),
        out_shape=jax.ShapeDtypeStruct((_L, _B // bb, 2, _LANES), jnp.int32),
    )(data_0, data_1, data_2, data_3)
    return out.reshape(_ROWS, 2, _LANES)


def kernel(data_0, data_1, data_2, data_3, word_table, tag_table, is_content):
    s = jnp.asarray(is_content, jnp.float32)
    idx_all = _tc_prep(data_0, data_1, data_2, data_3)
    e = jnp.arange(_EXT, dtype=jnp.int32)
    ext = jnp.concatenate([
        jnp.repeat(tag_table, 4, axis=0),
        (((e >> 1) & 1).astype(jnp.float32) * s)[:, None],
        ((e & 1).astype(jnp.float32) * s)[:, None],
        jnp.zeros((_EXT, _EPAD - _EDIM), jnp.float32),
    ], axis=1)
    out_fm = _sc_gather(word_table, ext.reshape(-1), idx_all)
    return jnp.transpose(out_fm, (1, 2, 0))


# speedup vs baseline: 1.4705x; 1.4705x over previous
import functools

import jax
import jax.numpy as jnp
from jax import lax
from jax.experimental import pallas as pl
from jax.experimental.pallas import tpu as pltpu
from jax.experimental.pallas import tpu_sc as plsc

_VOCAB = 100000
_DIM = 64
_B = 4096
_L = 200
_TAGS = 59
_TDIM = 4
_EDIM = _TDIM + 2
_EPAD = 8
_ODIM = _DIM + _EDIM
_EXT = _TAGS * 4

_N = _B * _L
_LANES = 256
_ROWS = _N // _LANES
_NC = 2
_NS = 16
_NW = _NC * _NS
_ROWS_PER_W = _ROWS // _NW
_CHUNK_ROWS = 2
_CHUNK = _CHUNK_ROWS * _LANES
_STEPS = _ROWS_PER_W // _CHUNK_ROWS


def _sc_gather(word_table, ext_table, idx_all):
    mesh = plsc.VectorSubcoreMesh(core_axis_name="c", subcore_axis_name="s")

    @functools.partial(
        pl.kernel,
        mesh=mesh,
        compiler_params=pltpu.CompilerParams(
            use_tc_tiling_on_sc=False, needs_layout_passes=False),
        out_type=[jax.ShapeDtypeStruct((_N, _DIM), jnp.float32),
                  jax.ShapeDtypeStruct((_N * _EPAD,), jnp.float32)],
        scratch_types=[
            pltpu.VMEM((_CHUNK_ROWS, 2, _LANES), jnp.int32),
            pltpu.VMEM((_CHUNK_ROWS, 2, _LANES), jnp.int32),
            pltpu.VMEM((_CHUNK, _DIM), jnp.float32),
            pltpu.VMEM((_CHUNK, _DIM), jnp.float32),
            pltpu.VMEM((_CHUNK * _EPAD,), jnp.float32),
            pltpu.VMEM((_CHUNK * _EPAD,), jnp.float32),
            pltpu.VMEM((_EXT * _EPAD,), jnp.float32),
            pltpu.SemaphoreType.DMA,
            pltpu.SemaphoreType.DMA,
            pltpu.SemaphoreType.DMA,
            pltpu.SemaphoreType.DMA,
        ],
    )
    def k(word_hbm, ext_hbm, idx_hbm, outw_hbm, oute_hbm,
          ibuf0, ibuf1, wbuf0, wbuf1, ebuf0, ebuf1, extv,
          gsem0, gsem1, ssem0, ssem1):
        wid = lax.axis_index("s") * _NC + lax.axis_index("c")
        row0 = wid * _ROWS_PER_W
        ibuf = (ibuf0, ibuf1)
        wbuf = (wbuf0, wbuf1)
        ebuf = (ebuf0, ebuf1)
        gsem = (gsem0, gsem1)
        ssem = (ssem0, ssem1)
        lane = lax.iota(jnp.int32, 16)

        def gather_copies(p):
            for b in range(_CHUNK_ROWS):
                yield pltpu.make_async_copy(
                    word_hbm.at[ibuf[p].at[b, 0]],
                    wbuf[p].at[pl.ds(b * _LANES, _LANES)],
                    gsem[p])

        def ext_compute(p):
            for b in range(_CHUNK_ROWS):
                for g in range(_LANES // 16):
                    i0 = b * _LANES + g * 16
                    e8 = ibuf[p][b, 1, pl.ds(g * 16, 16)] * _EPAD
                    pos8 = (lane + i0) * _EPAD
                    for c in range(_EDIM):
                        vals = plsc.load_gather(extv, [e8 + c])
                        plsc.store_scatter(ebuf[p], [pos8 + c], vals)

        def store_copies(p, r):
            base = r * _LANES
            yield pltpu.make_async_copy(
                wbuf[p], outw_hbm.at[pl.ds(base, _CHUNK)], ssem[p])
            yield pltpu.make_async_copy(
                ebuf[p], oute_hbm.at[pl.ds(base * _EPAD, _CHUNK * _EPAD)],
                ssem[p])

        pltpu.sync_copy(ext_hbm, extv)
        pltpu.sync_copy(idx_hbm.at[pl.ds(row0, _CHUNK_ROWS)], ibuf[0])
        for c in gather_copies(0):
            c.start()
        pltpu.sync_copy(
            idx_hbm.at[pl.ds(row0 + _CHUNK_ROWS, _CHUNK_ROWS)], ibuf[1])

        def step(j, p):
            r = row0 + j * _CHUNK_ROWS

            @pl.when(j >= 1)
            def _():
                for c in store_copies(1 - p, r):
                    c.wait()

            ext_compute(p)

            for c in gather_copies(p):
                c.wait()

            @pl.when(j + 1 < _STEPS)
            def _():
                for c in gather_copies(1 - p):
                    c.start()

            for c in store_copies(p, r):
                c.start()

            @pl.when(j + 2 < _STEPS)
            def _():
                pltpu.sync_copy(
                    idx_hbm.at[pl.ds(r + 2 * _CHUNK_ROWS, _CHUNK_ROWS)],
                    ibuf[p])

        def body(i, carry):
            step(2 * i, 0)
            step(2 * i + 1, 1)
            return carry

        lax.fori_loop(0, _STEPS // 2, body, 0)

        for c in store_copies(1, row0 + (_STEPS - 1) * _CHUNK_ROWS):
            c.wait()

    return k(word_table, ext_table, idx_all)


def _tc_prep(data_0, data_1, data_2, data_3):
    bb = _LANES

    def body(d0, d1, d2, d3, o):
        e = d1[...] * 4 + d2[...] * 2 + d3[...]
        o[:, 0, 0, :] = jnp.transpose(d0[...], (1, 0))
        o[:, 0, 1, :] = jnp.transpose(e, (1, 0))

    out = pl.pallas_call(
        body,
        grid=(_B // bb,),
        in_specs=[pl.BlockSpec((bb, _L), lambda j: (j, 0))] * 4,
        out_specs=pl.BlockSpec((_L, 1, 2, _LANES), lambda j: (0, j, 0, 0)),
        out_shape=jax.ShapeDtypeStruct((_L, _B // bb, 2, _LANES), jnp.int32),
    )(data_0, data_1, data_2, data_3)
    return out.reshape(_ROWS, 2, _LANES)


def kernel(data_0, data_1, data_2, data_3, word_table, tag_table, is_content):
    s = jnp.asarray(is_content, jnp.float32)
    idx_all = _tc_prep(data_0, data_1, data_2, data_3)
    e = jnp.arange(_EXT, dtype=jnp.int32)
    ext = jnp.concatenate([
        jnp.repeat(tag_table, 4, axis=0),
        (((e >> 1) & 1).astype(jnp.float32) * s)[:, None],
        ((e & 1).astype(jnp.float32) * s)[:, None],
        jnp.zeros((_EXT, _EPAD - _EDIM), jnp.float32),
    ], axis=1)
    out_w, out_e = _sc_gather(word_table, ext.reshape(-1), idx_all)
    return jnp.concatenate([
        out_w.reshape(_L, _B, _DIM),
        out_e.reshape(_L, _B, _EPAD)[:, :, :_EDIM],
    ], axis=2)


# speedup vs baseline: 1.8908x; 1.2858x over previous
import functools

import jax
import jax.numpy as jnp
from jax import lax
from jax.experimental import pallas as pl
from jax.experimental.pallas import tpu as pltpu
from jax.experimental.pallas import tpu_sc as plsc

_VOCAB = 100000
_DIM = 64
_B = 4096
_L = 200
_TAGS = 59
_TDIM = 4
_EDIM = _TDIM + 2
_EPAD = 8
_ODIM = _DIM + _EDIM
_EXT = _TAGS * 4

_N = _B * _L
_LANES = 256
_ROWS = _N // _LANES
_NC = 2
_NS = 16
_NW = _NC * _NS
_ROWS_PER_W = _ROWS // _NW
_CHUNK_ROWS = 2
_CHUNK = _CHUNK_ROWS * _LANES
_STEPS = _ROWS_PER_W // _CHUNK_ROWS


def _sc_gather(word_table, ext_table, idx_all):
    mesh = plsc.VectorSubcoreMesh(core_axis_name="c", subcore_axis_name="s")

    @functools.partial(
        pl.kernel,
        mesh=mesh,
        compiler_params=pltpu.CompilerParams(
            use_tc_tiling_on_sc=False, needs_layout_passes=False),
        out_type=[jax.ShapeDtypeStruct((_N, _DIM), jnp.float32),
                  jax.ShapeDtypeStruct((_EPAD, _L, _B), jnp.float32)],
        scratch_types=[
            pltpu.VMEM((_CHUNK_ROWS, 2, _LANES), jnp.int32),
            pltpu.VMEM((_CHUNK_ROWS, 2, _LANES), jnp.int32),
            pltpu.VMEM((_CHUNK, _DIM), jnp.float32),
            pltpu.VMEM((_CHUNK, _DIM), jnp.float32),
            pltpu.VMEM((_EPAD, 1, _CHUNK), jnp.float32),
            pltpu.VMEM((_EPAD, 1, _CHUNK), jnp.float32),
            pltpu.VMEM((_EXT * _EPAD,), jnp.float32),
            pltpu.SemaphoreType.DMA,
            pltpu.SemaphoreType.DMA,
            pltpu.SemaphoreType.DMA,
            pltpu.SemaphoreType.DMA,
        ],
    )
    def k(word_hbm, ext_hbm, idx_hbm, outw_hbm, oute_hbm,
          ibuf0, ibuf1, wbuf0, wbuf1, ebuf0, ebuf1, extv,
          gsem0, gsem1, ssem0, ssem1):
        wid = lax.axis_index("s") * _NC + lax.axis_index("c")
        row0 = wid * _ROWS_PER_W
        ibuf = (ibuf0, ibuf1)
        wbuf = (wbuf0, wbuf1)
        ebuf = (ebuf0, ebuf1)
        gsem = (gsem0, gsem1)
        ssem = (ssem0, ssem1)
        lane = lax.iota(jnp.int32, 16)

        def gather_copies(p):
            for b in range(_CHUNK_ROWS):
                yield pltpu.make_async_copy(
                    word_hbm.at[ibuf[p].at[b, 0]],
                    wbuf[p].at[pl.ds(b * _LANES, _LANES)],
                    gsem[p])

        def ext_compute(p):
            for b in range(_CHUNK_ROWS):
                for g in range(_LANES // 16):
                    i0 = b * _LANES + g * 16
                    e8 = ibuf[p][b, 1, pl.ds(g * 16, 16)] * _EPAD
                    for c in range(_EDIM):
                        vals = plsc.load_gather(extv, [e8 + c])
                        ebuf[p][c, 0, pl.ds(i0, 16)] = vals

        def store_copies(p, r):
            base = r * _LANES
            l = base // _B
            b0 = lax.rem(base, _B)
            yield pltpu.make_async_copy(
                wbuf[p], outw_hbm.at[pl.ds(base, _CHUNK)], ssem[p])
            yield pltpu.make_async_copy(
                ebuf[p],
                oute_hbm.at[:, pl.ds(l, 1), pl.ds(b0, _CHUNK)],
                ssem[p])

        pltpu.sync_copy(ext_hbm, extv)
        pltpu.sync_copy(idx_hbm.at[pl.ds(row0, _CHUNK_ROWS)], ibuf[0])
        for c in gather_copies(0):
            c.start()
        pltpu.sync_copy(
            idx_hbm.at[pl.ds(row0 + _CHUNK_ROWS, _CHUNK_ROWS)], ibuf[1])

        def step(j, p):
            r = row0 + j * _CHUNK_ROWS

            @pl.when(j >= 1)
            def _():
                for c in store_copies(1 - p, r):
                    c.wait()

            ext_compute(p)

            for c in gather_copies(p):
                c.wait()

            @pl.when(j + 1 < _STEPS)
            def _():
                for c in gather_copies(1 - p):
                    c.start()

            for c in store_copies(p, r):
                c.start()

            @pl.when(j + 2 < _STEPS)
            def _():
                pltpu.sync_copy(
                    idx_hbm.at[pl.ds(r + 2 * _CHUNK_ROWS, _CHUNK_ROWS)],
                    ibuf[p])

        def body(i, carry):
            step(2 * i, 0)
            step(2 * i + 1, 1)
            return carry

        lax.fori_loop(0, _STEPS // 2, body, 0)

        for c in store_copies(1, row0 + (_STEPS - 1) * _CHUNK_ROWS):
            c.wait()

    return k(word_table, ext_table, idx_all)


def _tc_prep(data_0, data_1, data_2, data_3):
    bb = _LANES

    def body(d0, d1, d2, d3, o):
        e = d1[...] * 4 + d2[...] * 2 + d3[...]
        o[:, 0, 0, :] = jnp.transpose(d0[...], (1, 0))
        o[:, 0, 1, :] = jnp.transpose(e, (1, 0))

    out = pl.pallas_call(
        body,
        grid=(_B // bb,),
        in_specs=[pl.BlockSpec((bb, _L), lambda j: (j, 0))] * 4,
        out_specs=pl.BlockSpec((_L, 1, 2, _LANES), lambda j: (0, j, 0, 0)),
        out_shape=jax.ShapeDtypeStruct((_L, _B // bb, 2, _LANES), jnp.int32),
    )(data_0, data_1, data_2, data_3)
    return out.reshape(_ROWS, 2, _LANES)


def kernel(data_0, data_1, data_2, data_3, word_table, tag_table, is_content):
    s = jnp.asarray(is_content, jnp.float32)
    idx_all = _tc_prep(data_0, data_1, data_2, data_3)
    e = jnp.arange(_EXT, dtype=jnp.int32)
    ext = jnp.concatenate([
        jnp.repeat(tag_table, 4, axis=0),
        (((e >> 1) & 1).astype(jnp.float32) * s)[:, None],
        ((e & 1).astype(jnp.float32) * s)[:, None],
        jnp.zeros((_EXT, _EPAD - _EDIM), jnp.float32),
    ], axis=1)
    out_w, out_e = _sc_gather(word_table, ext.reshape(-1), idx_all)
    return jnp.concatenate([
        out_w.reshape(_L, _B, _DIM),
        jnp.transpose(out_e[:_EDIM], (1, 2, 0)),
    ], axis=2)


# speedup vs baseline: 2.3214x; 1.2277x over previous
import functools

import jax
import jax.numpy as jnp
from jax import lax
from jax.experimental import pallas as pl
from jax.experimental.pallas import tpu as pltpu
from jax.experimental.pallas import tpu_sc as plsc

_VOCAB = 100000
_DIM = 64
_B = 4096
_L = 200
_TAGS = 59
_TDIM = 4
_EDIM = _TDIM + 2
_EPAD = 8
_ODIM = _DIM + _EDIM
_EXT = _TAGS * 4

_WPAD = 128
_N = _B * _L
_LANES = 256
_ROWS = _N // _LANES
_NC = 2
_NS = 16
_NW = _NC * _NS
_ROWS_PER_W = _ROWS // _NW
_CHUNK_ROWS = 1
_CHUNK = _CHUNK_ROWS * _LANES
_STEPS = _ROWS_PER_W // _CHUNK_ROWS


def _sc_gather(word_table, ext_table, idx_all):
    mesh = plsc.VectorSubcoreMesh(core_axis_name="c", subcore_axis_name="s")

    @functools.partial(
        pl.kernel,
        mesh=mesh,
        compiler_params=pltpu.CompilerParams(
            use_tc_tiling_on_sc=False, needs_layout_passes=False),
        out_type=[jax.ShapeDtypeStruct((_N, _WPAD), jnp.float32),
                  jax.ShapeDtypeStruct((_EPAD, _L, _B), jnp.float32)],
        scratch_types=[
            pltpu.VMEM((_CHUNK_ROWS, 2, _LANES), jnp.int32),
            pltpu.VMEM((_CHUNK_ROWS, 2, _LANES), jnp.int32),
            pltpu.VMEM((_CHUNK, _WPAD), jnp.float32),
            pltpu.VMEM((_CHUNK, _WPAD), jnp.float32),
            pltpu.VMEM((_EPAD, 1, _CHUNK), jnp.float32),
            pltpu.VMEM((_EPAD, 1, _CHUNK), jnp.float32),
            pltpu.VMEM((_EXT * _EPAD,), jnp.float32),
            pltpu.SemaphoreType.DMA,
            pltpu.SemaphoreType.DMA,
            pltpu.SemaphoreType.DMA,
            pltpu.SemaphoreType.DMA,
        ],
    )
    def k(word_hbm, ext_hbm, idx_hbm, outw_hbm, oute_hbm,
          ibuf0, ibuf1, wbuf0, wbuf1, ebuf0, ebuf1, extv,
          gsem0, gsem1, ssem0, ssem1):
        wid = lax.axis_index("s") * _NC + lax.axis_index("c")
        row0 = wid * _ROWS_PER_W
        ibuf = (ibuf0, ibuf1)
        wbuf = (wbuf0, wbuf1)
        ebuf = (ebuf0, ebuf1)
        gsem = (gsem0, gsem1)
        ssem = (ssem0, ssem1)
        lane = lax.iota(jnp.int32, 16)

        def gather_copies(p):
            for b in range(_CHUNK_ROWS):
                yield pltpu.make_async_copy(
                    word_hbm.at[ibuf[p].at[b, 0]],
                    wbuf[p].at[pl.ds(b * _LANES, _LANES)],
                    gsem[p])

        def ext_compute(p):
            for b in range(_CHUNK_ROWS):
                for g in range(_LANES // 16):
                    i0 = b * _LANES + g * 16
                    e8 = ibuf[p][b, 1, pl.ds(g * 16, 16)] * _EPAD
                    for c in range(_EDIM):
                        vals = plsc.load_gather(extv, [e8 + c])
                        ebuf[p][c, 0, pl.ds(i0, 16)] = vals

        def store_copies(p, r):
            base = r * _LANES
            l = base // _B
            b0 = lax.rem(base, _B)
            yield pltpu.make_async_copy(
                wbuf[p], outw_hbm.at[pl.ds(base, _CHUNK)], ssem[p])
            yield pltpu.make_async_copy(
                ebuf[p],
                oute_hbm.at[:, pl.ds(l, 1), pl.ds(b0, _CHUNK)],
                ssem[p])

        pltpu.sync_copy(ext_hbm, extv)
        pltpu.sync_copy(idx_hbm.at[pl.ds(row0, _CHUNK_ROWS)], ibuf[0])
        for c in gather_copies(0):
            c.start()
        pltpu.sync_copy(
            idx_hbm.at[pl.ds(row0 + _CHUNK_ROWS, _CHUNK_ROWS)], ibuf[1])

        def step(j, p):
            r = row0 + j * _CHUNK_ROWS

            @pl.when(j >= 1)
            def _():
                for c in store_copies(1 - p, r):
                    c.wait()

            ext_compute(p)

            for c in gather_copies(p):
                c.wait()

            @pl.when(j + 1 < _STEPS)
            def _():
                for c in gather_copies(1 - p):
                    c.start()

            for c in store_copies(p, r):
                c.start()

            @pl.when(j + 2 < _STEPS)
            def _():
                pltpu.sync_copy(
                    idx_hbm.at[pl.ds(r + 2 * _CHUNK_ROWS, _CHUNK_ROWS)],
                    ibuf[p])

        def body(i, carry):
            step(2 * i, 0)
            step(2 * i + 1, 1)
            return carry

        lax.fori_loop(0, _STEPS // 2, body, 0)

        for c in store_copies(1, row0 + (_STEPS - 1) * _CHUNK_ROWS):
            c.wait()

    return k(word_table, ext_table, idx_all)


def _tc_prep(data_0, data_1, data_2, data_3):
    bb = _LANES

    def body(d0, d1, d2, d3, o):
        e = d1[...] * 4 + d2[...] * 2 + d3[...]
        o[:, 0, 0, :] = jnp.transpose(d0[...], (1, 0))
        o[:, 0, 1, :] = jnp.transpose(e, (1, 0))

    out = pl.pallas_call(
        body,
        grid=(_B // bb,),
        in_specs=[pl.BlockSpec((bb, _L), lambda j: (j, 0))] * 4,
        out_specs=pl.BlockSpec((_L, 1, 2, _LANES), lambda j: (0, j, 0, 0)),
        out_shape=jax.ShapeDtypeStruct((_L, _B // bb, 2, _LANES), jnp.int32),
    )(data_0, data_1, data_2, data_3)
    return out.reshape(_ROWS, 2, _LANES)


def kernel(data_0, data_1, data_2, data_3, word_table, tag_table, is_content):
    s = jnp.asarray(is_content, jnp.float32)
    idx_all = _tc_prep(data_0, data_1, data_2, data_3)
    e = jnp.arange(_EXT, dtype=jnp.int32)
    ext = jnp.concatenate([
        jnp.repeat(tag_table, 4, axis=0),
        (((e >> 1) & 1).astype(jnp.float32) * s)[:, None],
        ((e & 1).astype(jnp.float32) * s)[:, None],
        jnp.zeros((_EXT, _EPAD - _EDIM), jnp.float32),
    ], axis=1)
    wt128 = jnp.pad(word_table, ((0, 0), (0, _WPAD - _DIM)))
    out_w, out_e = _sc_gather(wt128, ext.reshape(-1), idx_all)
    return jnp.concatenate([
        out_w.reshape(_L, _B, _WPAD)[:, :, :_DIM],
        jnp.transpose(out_e[:_EDIM], (1, 2, 0)),
    ], axis=2)


# speedup vs baseline: 2.3294x; 1.0034x over previous
import functools

import jax
import jax.numpy as jnp
from jax import lax
from jax.experimental import pallas as pl
from jax.experimental.pallas import tpu as pltpu
from jax.experimental.pallas import tpu_sc as plsc

_VOCAB = 100000
_DIM = 64
_B = 4096
_L = 200
_TAGS = 59
_TDIM = 4
_EDIM = _TDIM + 2
_EPAD = 8
_ODIM = _DIM + _EDIM
_EXT = _TAGS * 4

_WPAD = 128
_N = _B * _L
_LANES = 256
_ROWS = _N // _LANES
_NC = 2
_NS = 16
_NW = _NC * _NS
_ROWS_PER_W = _ROWS // _NW
_CHUNK_ROWS = 1
_CHUNK = _CHUNK_ROWS * _LANES
_STEPS = _ROWS_PER_W // _CHUNK_ROWS


def _sc_gather(word_table, ext_table, idx_all):
    mesh = plsc.VectorSubcoreMesh(core_axis_name="c", subcore_axis_name="s")

    @functools.partial(
        pl.kernel,
        mesh=mesh,
        compiler_params=pltpu.CompilerParams(
            use_tc_tiling_on_sc=False, needs_layout_passes=False),
        out_type=[jax.ShapeDtypeStruct((_N, _WPAD), jnp.float32),
                  jax.ShapeDtypeStruct((_EPAD, _L, _B), jnp.float32)],
        scratch_types=[
            pltpu.VMEM((_CHUNK_ROWS, 2 * _LANES), jnp.int32),
            pltpu.VMEM((_CHUNK_ROWS, 2 * _LANES), jnp.int32),
            pltpu.VMEM((_CHUNK, _WPAD), jnp.float32),
            pltpu.VMEM((_CHUNK, _WPAD), jnp.float32),
            pltpu.VMEM((_EPAD, 1, _CHUNK), jnp.float32),
            pltpu.VMEM((_EPAD, 1, _CHUNK), jnp.float32),
            pltpu.VMEM((_EXT * _EPAD,), jnp.float32),
            pltpu.SemaphoreType.DMA,
            pltpu.SemaphoreType.DMA,
            pltpu.SemaphoreType.DMA,
            pltpu.SemaphoreType.DMA,
        ],
    )
    def k(word_hbm, ext_hbm, idx_hbm, outw_hbm, oute_hbm,
          ibuf0, ibuf1, wbuf0, wbuf1, ebuf0, ebuf1, extv,
          gsem0, gsem1, ssem0, ssem1):
        wid = lax.axis_index("s") * _NC + lax.axis_index("c")
        row0 = wid * _ROWS_PER_W
        jblk0 = wid * _ROWS_PER_W // _L
        l0 = wid * _ROWS_PER_W % _L
        ibuf = (ibuf0, ibuf1)
        wbuf = (wbuf0, wbuf1)
        ebuf = (ebuf0, ebuf1)
        gsem = (gsem0, gsem1)
        ssem = (ssem0, ssem1)
        lane = lax.iota(jnp.int32, 16)

        def gather_copies(p):
            for b in range(_CHUNK_ROWS):
                yield pltpu.make_async_copy(
                    word_hbm.at[ibuf[p].at[b, pl.ds(0, _LANES)]],
                    wbuf[p].at[pl.ds(b * _LANES, _LANES)],
                    gsem[p])

        def ext_compute(p):
            for b in range(_CHUNK_ROWS):
                for g in range(_LANES // 16):
                    i0 = b * _LANES + g * 16
                    e8 = ibuf[p][b, pl.ds(_LANES + g * 16, 16)] * _EPAD
                    for c in range(_EDIM):
                        vals = plsc.load_gather(extv, [e8 + c])
                        ebuf[p][c, 0, pl.ds(i0, 16)] = vals

        def store_copies(p, r):
            l = l0 + lax.rem(r, _ROWS_PER_W)
            b0 = jblk0 * _LANES
            base = l * _B + b0
            yield pltpu.make_async_copy(
                wbuf[p], outw_hbm.at[pl.ds(base, _CHUNK)], ssem[p])
            yield pltpu.make_async_copy(
                ebuf[p],
                oute_hbm.at[:, pl.ds(l, 1), pl.ds(b0, _CHUNK)],
                ssem[p])

        pltpu.sync_copy(ext_hbm, extv)
        pltpu.sync_copy(idx_hbm.at[pl.ds(row0, _CHUNK_ROWS)], ibuf[0])
        for c in gather_copies(0):
            c.start()
        pltpu.sync_copy(
            idx_hbm.at[pl.ds(row0 + _CHUNK_ROWS, _CHUNK_ROWS)], ibuf[1])

        def step(j, p):
            r = row0 + j * _CHUNK_ROWS

            @pl.when(j >= 1)
            def _():
                for c in store_copies(1 - p, r):
                    c.wait()

            ext_compute(p)

            for c in gather_copies(p):
                c.wait()

            @pl.when(j + 1 < _STEPS)
            def _():
                for c in gather_copies(1 - p):
                    c.start()

            for c in store_copies(p, r):
                c.start()

            @pl.when(j + 2 < _STEPS)
            def _():
                pltpu.sync_copy(
                    idx_hbm.at[pl.ds(r + 2 * _CHUNK_ROWS, _CHUNK_ROWS)],
                    ibuf[p])

        def body(i, carry):
            step(2 * i, 0)
            step(2 * i + 1, 1)
            return carry

        lax.fori_loop(0, _STEPS // 2, body, 0)

        for c in store_copies(1, row0 + (_STEPS - 1) * _CHUNK_ROWS):
            c.wait()

    return k(word_table, ext_table, idx_all)


def _tc_prep(data_0, data_1, data_2, data_3):
    bb = _LANES

    def body(d0, d1, d2, d3, o):
        e = d1[...] * 4 + d2[...] * 2 + d3[...]
        o[0, :, 0:bb] = jnp.transpose(d0[...], (1, 0))
        o[0, :, bb:2 * bb] = jnp.transpose(e, (1, 0))

    out = pl.pallas_call(
        body,
        grid=(_B // bb,),
        in_specs=[pl.BlockSpec((bb, _L), lambda j: (j, 0))] * 4,
        out_specs=pl.BlockSpec((1, _L, 2 * _LANES), lambda j: (j, 0, 0)),
        out_shape=jax.ShapeDtypeStruct((_B // bb, _L, 2 * _LANES), jnp.int32),
    )(data_0, data_1, data_2, data_3)
    return out.reshape(_ROWS, 2 * _LANES)


def kernel(data_0, data_1, data_2, data_3, word_table, tag_table, is_content):
    s = jnp.asarray(is_content, jnp.float32)
    idx_all = _tc_prep(data_0, data_1, data_2, data_3)
    e = jnp.arange(_EXT, dtype=jnp.int32)
    ext = jnp.concatenate([
        jnp.repeat(tag_table, 4, axis=0),
        (((e >> 1) & 1).astype(jnp.float32) * s)[:, None],
        ((e & 1).astype(jnp.float32) * s)[:, None],
        jnp.zeros((_EXT, _EPAD - _EDIM), jnp.float32),
    ], axis=1)
    wt128 = jnp.pad(word_table, ((0, 0), (0, _WPAD - _DIM)))
    out_w, out_e = _sc_gather(wt128, ext.reshape(-1), idx_all)
    return jnp.concatenate([
        out_w.reshape(_L, _B, _WPAD)[:, :, :_DIM],
        jnp.transpose(out_e[:_EDIM], (1, 2, 0)),
    ], axis=2)
